# Initial kernel scaffold; baseline (speedup 1.0000x reference)
import jax
import jax.numpy as jnp
from jax.experimental import pallas as pl


def kernel(dense_input, sparse_input, emb, Wb1, Wb2, Wb3, Wb4, Wt1, Wt2, Wt3):
    raise NotImplementedError("write your pallas kernel here")


# speedup vs baseline: 3.8701x; 3.8701x over previous
import functools

import jax
import jax.numpy as jnp
import numpy as np
from jax import lax
from jax.experimental import pallas as pl
from jax.experimental.pallas import tpu as pltpu
from jax.experimental.pallas import tpu_sc as plsc

_B = 4096
_D = 64
_NS = 26
_NI = _NS + 1
_V = 4823

_LI = np.array([i for i in range(_NI) for j in range(i)])
_LJ = np.array([j for i in range(_NI) for j in range(i)])

_NC = 2
_NSC = 16
_NW = _NC * _NSC
_SPW = _B // _NW
_RPW = _SPW * _NS
_CH = 128
_NCH = _RPW // _CH

_sc_mesh = plsc.VectorSubcoreMesh(core_axis_name="c", subcore_axis_name="s")


@functools.partial(
    pl.kernel,
    mesh=_sc_mesh,
    compiler_params=pltpu.CompilerParams(use_tc_tiling_on_sc=False),
    out_type=jax.ShapeDtypeStruct((_B * _NS, _D), jnp.float32),
    scratch_types=[
        pltpu.VMEM((_NCH, _CH), jnp.int32),
        pltpu.VMEM((_CH, _D), jnp.float32),
        pltpu.VMEM((_CH, _D), jnp.float32),
        pltpu.SemaphoreType.DMA,
        pltpu.SemaphoreType.DMA,
        pltpu.SemaphoreType.DMA,
        pltpu.SemaphoreType.DMA,
    ],
)
def _sc_gather(idx_hbm, table_hbm, out_hbm, idx_v, buf0, buf1, g0, g1, s0, s1):
    wid = lax.axis_index("s") * _NC + lax.axis_index("c")
    base = wid * _RPW
    pltpu.sync_copy(idx_hbm.at[wid], idx_v)

    def body(h, carry):
        j0 = 2 * h
        j1 = 2 * h + 1
        c0 = pltpu.async_copy(table_hbm.at[idx_v.at[j0]], buf0, g0)
        c1 = pltpu.async_copy(table_hbm.at[idx_v.at[j1]], buf1, g1)
        c0.wait()
        w0 = pltpu.async_copy(buf0, out_hbm.at[pl.ds(base + j0 * _CH, _CH)], s0)
        c1.wait()
        w1 = pltpu.async_copy(buf1, out_hbm.at[pl.ds(base + j1 * _CH, _CH)], s1)
        w0.wait()
        w1.wait()
        return carry

    lax.fori_loop(0, _NCH // 2, body, 0)


_BBLK = 256
_NBLK = _B // _BBLK


def _tc_body(dense, y2, wb1, wb2, wb3, wb4, wt1a, wt1z, wt2, wt3, out):
    f32 = jnp.float32
    x = dense[:]
    y1 = jnp.maximum(jnp.dot(x, wb1[:], preferred_element_type=f32), 0.0)
    y1 = jnp.maximum(jnp.dot(y1, wb2[:], preferred_element_type=f32), 0.0)
    y1 = jnp.maximum(jnp.dot(y1, wb3[:], preferred_element_type=f32), 0.0)
    y1 = jnp.dot(y1, wb4[:], preferred_element_type=f32)

    t3 = jnp.concatenate([y1.reshape(_BBLK, 1, _D), y2[:]], axis=1)
    z = lax.dot_general(
        t3, t3,
        dimension_numbers=(((2,), (2,)), ((0,), (0,))),
        preferred_element_type=f32,
    )
    zf = z.reshape(_BBLK, _NI * _NI)

    h = jnp.dot(y1, wt1a[:], preferred_element_type=f32)
    h = h + jnp.dot(zf, wt1z[:], preferred_element_type=f32)
    h = jnp.maximum(h, 0.0)
    h = jnp.maximum(jnp.dot(h, wt2[:], preferred_element_type=f32), 0.0)
    out[:] = jax.nn.sigmoid(jnp.dot(h, wt3[:], preferred_element_type=f32))


def _const_spec(shape):
    return pl.BlockSpec(shape, lambda b: (0,) * len(shape))


_tc_call = pl.pallas_call(
    _tc_body,
    grid=(_NBLK,),
    in_specs=[
        pl.BlockSpec((_BBLK, 13), lambda b: (b, 0)),
        pl.BlockSpec((_BBLK, _NS, _D), lambda b: (b, 0, 0)),
        _const_spec((13, 512)),
        _const_spec((512, 256)),
        _const_spec((256, 64)),
        _const_spec((64, _D)),
        _const_spec((_D, 512)),
        _const_spec((_NI * _NI, 512)),
        _const_spec((512, 256)),
        _const_spec((256, 1)),
    ],
    out_specs=pl.BlockSpec((_BBLK, 1), lambda b: (b, 0)),
    out_shape=jax.ShapeDtypeStruct((_B, 1), jnp.float32),
)


def kernel(dense_input, sparse_input, emb, Wb1, Wb2, Wb3, Wb4, Wt1, Wt2, Wt3):
    idx = sparse_input.astype(jnp.int32).reshape(_NW, _NCH, _CH)
    y2 = _sc_gather(idx, emb).reshape(_B, _NS, _D)

    pair_pos = jnp.asarray(_LI * _NI + _LJ, dtype=jnp.int32)
    wt1z = jnp.zeros((_NI * _NI, 512), jnp.float32).at[pair_pos].set(Wt1[_D:])
    wt1a = Wt1[:_D]

    return _tc_call(dense_input, y2, Wb1, Wb2, Wb3, Wb4, wt1a, wt1z, Wt2, Wt3)


# speedup vs baseline: 3.9038x; 1.0087x over previous
import functools

import jax
import jax.numpy as jnp
import numpy as np
from jax import lax
from jax.experimental import pallas as pl
from jax.experimental.pallas import tpu as pltpu
from jax.experimental.pallas import tpu_sc as plsc

_B = 4096
_D = 64
_NS = 26
_NI = _NS + 1
_V = 4823
_NSPLIT = 2
_BH = _B // _NSPLIT

_LI = np.array([i for i in range(_NI) for j in range(i)])
_LJ = np.array([j for i in range(_NI) for j in range(i)])

_NC = 2
_NSC = 16
_NW = _NC * _NSC
_CH = 128

_sc_mesh = plsc.VectorSubcoreMesh(core_axis_name="c", subcore_axis_name="s")


def _make_sc_gather(nb):
    rpw = nb * _NS // _NW
    nch = rpw // _CH
    assert rpw % _CH == 0

    @functools.partial(
        pl.kernel,
        mesh=_sc_mesh,
        compiler_params=pltpu.CompilerParams(use_tc_tiling_on_sc=False),
        out_type=jax.ShapeDtypeStruct((nb * _NS, _D), jnp.float32),
        scratch_types=[
            pltpu.VMEM((nch, _CH), jnp.int32),
            pltpu.VMEM((_CH, _D), jnp.float32),
            pltpu.VMEM((_CH, _D), jnp.float32),
            pltpu.SemaphoreType.DMA,
            pltpu.SemaphoreType.DMA,
            pltpu.SemaphoreType.DMA,
            pltpu.SemaphoreType.DMA,
        ],
    )
    def sc_gather(idx_hbm, table_hbm, out_hbm, idx_v, buf0, buf1, g0, g1, s0, s1):
        wid = lax.axis_index("s") * _NC + lax.axis_index("c")
        base = wid * rpw
        pltpu.sync_copy(idx_hbm.at[wid], idx_v)

        def body(h, carry):
            j0 = 2 * h
            j1 = 2 * h + 1
            c0 = pltpu.async_copy(table_hbm.at[idx_v.at[j0]], buf0, g0)
            c1 = pltpu.async_copy(table_hbm.at[idx_v.at[j1]], buf1, g1)
            c0.wait()
            w0 = pltpu.async_copy(buf0, out_hbm.at[pl.ds(base + j0 * _CH, _CH)], s0)
            c1.wait()
            w1 = pltpu.async_copy(buf1, out_hbm.at[pl.ds(base + j1 * _CH, _CH)], s1)
            w0.wait()
            w1.wait()
            return carry

        lax.fori_loop(0, nch // 2, body, 0)
        if nch % 2:
            j = nch - 1
            c = pltpu.async_copy(table_hbm.at[idx_v.at[j]], buf0, g0)
            c.wait()
            w = pltpu.async_copy(buf0, out_hbm.at[pl.ds(base + j * _CH, _CH)], s0)
            w.wait()

    return sc_gather


_sc_gather_half = _make_sc_gather(_BH)

_BBLK = 256
_NBLK = _BH // _BBLK


def _tc_body(dense, y2, wb1, wb2, wb3, wb4, wt1a, wt1z, wt2, wt3, out):
    f32 = jnp.float32
    x = dense[:]
    y1 = jnp.maximum(jnp.dot(x, wb1[:], preferred_element_type=f32), 0.0)
    y1 = jnp.maximum(jnp.dot(y1, wb2[:], preferred_element_type=f32), 0.0)
    y1 = jnp.maximum(jnp.dot(y1, wb3[:], preferred_element_type=f32), 0.0)
    y1 = jnp.dot(y1, wb4[:], preferred_element_type=f32)

    t3 = jnp.concatenate([y1.reshape(_BBLK, 1, _D), y2[:]], axis=1)
    z = lax.dot_general(
        t3, t3,
        dimension_numbers=(((2,), (2,)), ((0,), (0,))),
        preferred_element_type=f32,
    )
    zf = z.reshape(_BBLK, _NI * _NI)

    h = jnp.dot(y1, wt1a[:], preferred_element_type=f32)
    h = h + jnp.dot(zf, wt1z[:], preferred_element_type=f32)
    h = jnp.maximum(h, 0.0)
    h = jnp.maximum(jnp.dot(h, wt2[:], preferred_element_type=f32), 0.0)
    out[:] = jax.nn.sigmoid(jnp.dot(h, wt3[:], preferred_element_type=f32))


def _const_spec(shape):
    return pl.BlockSpec(shape, lambda b: (0,) * len(shape))


_tc_call = pl.pallas_call(
    _tc_body,
    grid=(_NBLK,),
    in_specs=[
        pl.BlockSpec((_BBLK, 13), lambda b: (b, 0)),
        pl.BlockSpec((_BBLK, _NS, _D), lambda b: (b, 0, 0)),
        _const_spec((13, 512)),
        _const_spec((512, 256)),
        _const_spec((256, 64)),
        _const_spec((64, _D)),
        _const_spec((_D, 512)),
        _const_spec((_NI * _NI, 512)),
        _const_spec((512, 256)),
        _const_spec((256, 1)),
    ],
    out_specs=pl.BlockSpec((_BBLK, 1), lambda b: (b, 0)),
    out_shape=jax.ShapeDtypeStruct((_BH, 1), jnp.float32),
)


def kernel(dense_input, sparse_input, emb, Wb1, Wb2, Wb3, Wb4, Wt1, Wt2, Wt3):
    pair_pos = jnp.asarray(_LI * _NI + _LJ, dtype=jnp.int32)
    wt1z = jnp.zeros((_NI * _NI, 512), jnp.float32).at[pair_pos].set(Wt1[_D:])
    wt1a = Wt1[:_D]

    idx = sparse_input.astype(jnp.int32)
    outs = []
    for h in range(_NSPLIT):
        idx_h = idx[h * _BH:(h + 1) * _BH].reshape(_NW, -1, _CH)
        y2_h = _sc_gather_half(idx_h, emb).reshape(_BH, _NS, _D)
        dense_h = dense_input[h * _BH:(h + 1) * _BH]
        outs.append(_tc_call(dense_h, y2_h, Wb1, Wb2, Wb3, Wb4,
                             wt1a, wt1z, Wt2, Wt3))
    return jnp.concatenate(outs, axis=0)
